# Initial kernel scaffold; baseline (speedup 1.0000x reference)
import jax
import jax.numpy as jnp
from jax.experimental import pallas as pl


def kernel(scalar_feats, coords, E_idx, W_e3, b_e3, W_att1, b_att1, W_att2, b_att2, W_ffn1, b_ffn1, W_ffn2, b_ffn2, ln_g, ln_b):
    raise NotImplementedError("write your pallas kernel here")


# speedup vs baseline: 3.6300x; 3.6300x over previous
import functools

import jax
import jax.numpy as jnp
from jax import lax
from jax.experimental import pallas as pl
from jax.experimental.pallas import tpu as pltpu
from jax.experimental.pallas import tpu_sc as plsc

CS = 128
VD = 16
VROW = 128
CPW = 128


def _gelu_exact(x):
    return 0.5 * x * (1.0 + lax.erf(x * 0.7071067811865476))


def _prepass_body(sf_ref, co_ref, wcat_ref, bcat_ref, ts_ref, tv_ref, cproj_ref):
    lin = jnp.dot(sf_ref[...], wcat_ref[...],
                  preferred_element_type=jnp.float32) + bcat_ref[...]
    t = sf_ref.shape[0]
    ts_ref[...] = lin[:, 0:CS]
    tv_ref[:, 0:48] = lin[:, CS:176]
    tv_ref[:, 48:51] = co_ref[...]
    tv_ref[:, 51:VROW] = jnp.zeros((t, VROW - 51), jnp.float32)
    cproj_ref[...] = lin[:, 176:304]


def _prepass_call(sf, coords, wcat, bcat, tile):
    n = sf.shape[0]
    grid = n // tile
    return pl.pallas_call(
        _prepass_body,
        grid=(grid,),
        in_specs=[
            pl.BlockSpec((tile, CS), lambda i: (i, 0)),
            pl.BlockSpec((tile, 3), lambda i: (i, 0)),
            pl.BlockSpec((CS, 304), lambda i: (0, 0)),
            pl.BlockSpec((1, 304), lambda i: (0, 0)),
        ],
        out_specs=[
            pl.BlockSpec((tile, CS), lambda i: (i, 0)),
            pl.BlockSpec((tile, VROW), lambda i: (i, 0)),
            pl.BlockSpec((tile, CPW), lambda i: (i, 0)),
        ],
        out_shape=[
            jax.ShapeDtypeStruct((n, CS), jnp.float32),
            jax.ShapeDtypeStruct((n, VROW), jnp.float32),
            jax.ShapeDtypeStruct((n, CPW), jnp.float32),
        ],
    )(sf, coords, wcat, bcat)


_NC = 2
_NS = 16
_NW = _NC * _NS
_CHUNK = 80


def _make_gather(ne):
    per_w = ne // _NW
    n_chunks = per_w // _CHUNK
    mesh = plsc.VectorSubcoreMesh(core_axis_name="c", subcore_axis_name="s")

    @functools.partial(
        pl.kernel, mesh=mesh,
        out_type=[
            jax.ShapeDtypeStruct((ne, CS), jnp.float32),
            jax.ShapeDtypeStruct((ne, VROW), jnp.float32),
        ],
        scratch_types=[
            pltpu.VMEM((_CHUNK,), jnp.int32),
            pltpu.VMEM((_CHUNK, CS), jnp.float32),
            pltpu.VMEM((_CHUNK, VROW), jnp.float32),
            pltpu.SemaphoreType.DMA,
            pltpu.SemaphoreType.DMA,
        ],
    )
    def gather(ts_hbm, tv_hbm, idx_hbm, outs_hbm, outv_hbm,
               idx_v, rs_v, rv_v, sem_s, sem_v):
        wid = lax.axis_index("s") * _NC + lax.axis_index("c")
        base = wid * per_w

        def body(c, carry):
            off = pl.multiple_of(base + c * _CHUNK, 8)
            pltpu.sync_copy(idx_hbm.at[pl.ds(off, _CHUNK)], idx_v)
            cs_cp = pltpu.async_copy(ts_hbm.at[idx_v], rs_v, sem_s)
            cv_cp = pltpu.async_copy(tv_hbm.at[idx_v], rv_v, sem_v)
            cs_cp.wait()
            cv_cp.wait()
            pltpu.sync_copy(rs_v, outs_hbm.at[pl.ds(off, _CHUNK)])
            pltpu.sync_copy(rv_v, outv_hbm.at[pl.ds(off, _CHUNK)])
            return carry

        lax.fori_loop(0, n_chunks, body, 0)

    return gather


def _main_body(gs_ref, gv_ref, tab_ref, cp_ref, co_ref, w1n_ref, w1d_ref,
               w2_ref, b2_ref, wf1_ref, bf1_ref, wf2_ref, bf2_ref, lng_ref,
               lnb_ref, outs_ref, outv_ref, k_count):
    t = tab_ref.shape[0]
    cc = co_ref[...]
    cp = cp_ref[...]
    w1n = w1n_ref[...]
    w1d = w1d_ref[...]
    w2 = w2_ref[...]

    lane = lax.broadcasted_iota(jnp.int32, (t, k_count), 1)
    logits = jnp.zeros((t, k_count), jnp.float32)
    for k in range(k_count):
        nc = gv_ref[k][:, 48:51]
        diff = cc - nc
        dist = jnp.sqrt(jnp.sum(diff * diff, axis=1, keepdims=True) + 1e-6)
        h = cp + jnp.dot(gs_ref[k], w1n,
                         preferred_element_type=jnp.float32) + dist * w1d
        h = _gelu_exact(h)
        logit = jnp.sum(h * w2, axis=1, keepdims=True) + b2_ref[:, 0:1]
        logit = jnp.clip(logit, -10000.0, 10.0)
        logits = jnp.where(lane == k, logit, logits)

    m = jnp.max(logits, axis=1, keepdims=True)
    e = jnp.exp(logits - m)
    att = e / jnp.sum(e, axis=1, keepdims=True)

    smsg = jnp.zeros((t, CS), jnp.float32)
    vmsg = jnp.zeros((t, VD * 3), jnp.float32)
    for k in range(k_count):
        a = att[:, k:k + 1]
        smsg = smsg + a * gs_ref[k]
        vmsg = vmsg + a * gv_ref[k][:, 0:VD * 3]

    x = tab_ref[...] + smsg
    mu = jnp.mean(x, axis=1, keepdims=True)
    xc = x - mu
    var = jnp.mean(xc * xc, axis=1, keepdims=True)
    x = xc * jax.lax.rsqrt(var + 1e-5) * lng_ref[...] + lnb_ref[...]

    f = jnp.dot(x, wf1_ref[...], preferred_element_type=jnp.float32) + bf1_ref[...]
    f = _gelu_exact(f)
    f = jnp.dot(f, wf2_ref[...], preferred_element_type=jnp.float32) + bf2_ref[...]

    outs_ref[...] = x + f
    outv_ref[...] = vmsg


def _main_call(gs3, gv3, table_s, cproj, coords, w1n, w1d, w2, b2, wf1, bf1,
               wf2, bf2, lng, lnb, tile, k_count):
    n = table_s.shape[0]
    grid = n // tile
    const = lambda i: (0, 0)
    return pl.pallas_call(
        functools.partial(_main_body, k_count=k_count),
        grid=(grid,),
        in_specs=[
            pl.BlockSpec((k_count, tile, CS), lambda i: (0, i, 0)),
            pl.BlockSpec((k_count, tile, VROW), lambda i: (0, i, 0)),
            pl.BlockSpec((tile, CS), lambda i: (i, 0)),
            pl.BlockSpec((tile, CPW), lambda i: (i, 0)),
            pl.BlockSpec((tile, 3), lambda i: (i, 0)),
            pl.BlockSpec((CS, CS), const),
            pl.BlockSpec((1, CS), const),
            pl.BlockSpec((1, CS), const),
            pl.BlockSpec((1, CS), const),
            pl.BlockSpec((CS, 4 * CS), const),
            pl.BlockSpec((1, 4 * CS), const),
            pl.BlockSpec((4 * CS, CS), const),
            pl.BlockSpec((1, CS), const),
            pl.BlockSpec((1, CS), const),
            pl.BlockSpec((1, CS), const),
        ],
        out_specs=[
            pl.BlockSpec((tile, CS), lambda i: (i, 0)),
            pl.BlockSpec((tile, VD * 3), lambda i: (i, 0)),
        ],
        out_shape=[
            jax.ShapeDtypeStruct((n, CS), jnp.float32),
            jax.ShapeDtypeStruct((n, VD * 3), jnp.float32),
        ],
    )(gs3, gv3, table_s, cproj, coords, w1n, w1d, w2, b2, wf1, bf1, wf2, bf2,
      lng, lnb)


def kernel(scalar_feats, coords, E_idx, W_e3, b_e3, W_att1, b_att1, W_att2,
           b_att2, W_ffn1, b_ffn1, W_ffn2, b_ffn2, ln_g, ln_b):
    b, n, cs = scalar_feats.shape
    k_count = E_idx.shape[-1]
    sf = scalar_feats[0]
    co = coords[0]

    w1c = W_att1[:, 0:cs]
    w1n = W_att1[:, cs:2 * cs]
    w1d = W_att1[:, 2 * cs]
    we3s = W_e3[0:cs, :]
    wcat = jnp.concatenate([W_e3.T, (w1c @ we3s).T], axis=1)
    bcat = jnp.concatenate([b_e3, w1c @ b_e3[0:cs] + b_att1])[None, :]

    table_s, table_v, cproj = _prepass_call(sf, co, wcat, bcat, tile=1000)

    idx = jnp.transpose(E_idx[0]).reshape(-1).astype(jnp.int32)
    g_s, g_v = _make_gather(n * k_count)(table_s, table_v, idx)
    gs3 = g_s.reshape(k_count, n, CS)
    gv3 = g_v.reshape(k_count, n, VROW)

    outs, outv = _main_call(
        gs3, gv3, table_s, cproj, co,
        w1n.T, w1d[None, :], W_att2[0][None, :],
        jnp.broadcast_to(b_att2.reshape(1, 1), (1, CS)),
        W_ffn1.T, b_ffn1[None, :], W_ffn2.T, b_ffn2[None, :],
        ln_g[None, :], ln_b[None, :],
        tile=200, k_count=k_count,
    )
    return outs[None], outv.reshape(1, n, VD, 3)


# speedup vs baseline: 3.9128x; 1.0779x over previous
import functools

import jax
import jax.numpy as jnp
from jax import lax
from jax.experimental import pallas as pl
from jax.experimental.pallas import tpu as pltpu
from jax.experimental.pallas import tpu_sc as plsc

CS = 128
VD = 16
PW = 128
CPW = 128


def _gelu_exact(x):
    return 0.5 * x * (1.0 + lax.erf(x * 0.7071067811865476))


def _f32_to_f16bits(x):
    u = lax.bitcast_convert_type(x, jnp.uint32)
    s = (u >> jnp.uint32(16)) & jnp.uint32(0x8000)
    mag = u & jnp.uint32(0x7FFFFFFF)
    core = (mag - jnp.uint32(0x38000000) + jnp.uint32(0x1000)) >> jnp.uint32(13)
    core = jnp.where(mag < jnp.uint32(0x38800000), jnp.uint32(0), core)
    core = jnp.where(mag >= jnp.uint32(0x47800000), jnp.uint32(0x7BFF), core)
    return s | core


def _f16bits_to_f32(h):
    s = (h & jnp.uint32(0x8000)) << jnp.uint32(16)
    core = h & jnp.uint32(0x7FFF)
    mag = (core << jnp.uint32(13)) + jnp.uint32(0x38000000)
    mag = jnp.where(core == jnp.uint32(0), jnp.uint32(0), mag)
    return lax.bitcast_convert_type(s | mag, jnp.float32)


def _pack16(a, b):
    w = _f32_to_f16bits(a) | (_f32_to_f16bits(b) << jnp.uint32(16))
    return lax.bitcast_convert_type(w, jnp.float32)


def _unpack_lo(u):
    return _f16bits_to_f32(u & jnp.uint32(0xFFFF))


def _unpack_hi(u):
    return _f16bits_to_f32(u >> jnp.uint32(16))


def _prepass_body(sf_ref, co_ref, wcat_ref, bcat_ref, ts_ref, tp_ref, cproj_ref):
    lin = jnp.dot(sf_ref[...], wcat_ref[...],
                  preferred_element_type=jnp.float32) + bcat_ref[...]
    t = sf_ref.shape[0]
    s = lin[:, 0:CS]
    v = lin[:, CS:176]
    ts_ref[...] = s
    cproj_ref[...] = lin[:, 176:304]
    tp_ref[...] = jnp.concatenate(
        [_pack16(s[:, 0:64], s[:, 64:128]),
         _pack16(v[:, 0:24], v[:, 24:48]),
         co_ref[...],
         jnp.zeros((t, PW - 91), jnp.float32)], axis=1)


def _prepass_call(sf, coords, wcat, bcat, tile):
    n = sf.shape[0]
    grid = n // tile
    return pl.pallas_call(
        _prepass_body,
        grid=(grid,),
        in_specs=[
            pl.BlockSpec((tile, CS), lambda i: (i, 0)),
            pl.BlockSpec((tile, 3), lambda i: (i, 0)),
            pl.BlockSpec((CS, 304), lambda i: (0, 0)),
            pl.BlockSpec((1, 304), lambda i: (0, 0)),
        ],
        out_specs=[
            pl.BlockSpec((tile, CS), lambda i: (i, 0)),
            pl.BlockSpec((tile, PW), lambda i: (i, 0)),
            pl.BlockSpec((tile, CPW), lambda i: (i, 0)),
        ],
        out_shape=[
            jax.ShapeDtypeStruct((n, CS), jnp.float32),
            jax.ShapeDtypeStruct((n, PW), jnp.float32),
            jax.ShapeDtypeStruct((n, CPW), jnp.float32),
        ],
    )(sf, coords, wcat, bcat)


_NC = 2
_NS = 16
_NW = _NC * _NS
_CHUNK = 80


def _make_gather(ne):
    per_w = ne // _NW
    n_chunks = per_w // _CHUNK
    mesh = plsc.VectorSubcoreMesh(core_axis_name="c", subcore_axis_name="s")

    @functools.partial(
        pl.kernel, mesh=mesh,
        out_type=jax.ShapeDtypeStruct((ne, PW), jnp.float32),
        scratch_types=[
            pltpu.VMEM((_CHUNK,), jnp.int32),
            pltpu.VMEM((_CHUNK, PW), jnp.float32),
            pltpu.SemaphoreType.DMA,
        ],
    )
    def gather(tp_hbm, idx_hbm, outp_hbm, idx_v, rp_v, sem_p):
        wid = lax.axis_index("s") * _NC + lax.axis_index("c")
        base = wid * per_w

        def body(c, carry):
            off = pl.multiple_of(base + c * _CHUNK, 8)
            pltpu.sync_copy(idx_hbm.at[pl.ds(off, _CHUNK)], idx_v)
            cp_cp = pltpu.async_copy(tp_hbm.at[idx_v], rp_v, sem_p)
            cp_cp.wait()
            pltpu.sync_copy(rp_v, outp_hbm.at[pl.ds(off, _CHUNK)])
            return carry

        lax.fori_loop(0, n_chunks, body, 0)

    return gather


def _main_body(gp_ref, tab_ref, cp_ref, co_ref, w1n_ref, w1d_ref,
               w2_ref, b2_ref, wf1_ref, bf1_ref, wf2_ref, bf2_ref, lng_ref,
               lnb_ref, outs_ref, outv_ref, k_count):
    t = tab_ref.shape[0]
    cc = co_ref[...]
    cp = cp_ref[...]
    w1n_lo = w1n_ref[0:64, :]
    w1n_hi = w1n_ref[64:CS, :]
    w1d = w1d_ref[...]
    w2 = w2_ref[...]

    lane = lax.broadcasted_iota(jnp.int32, (t, k_count), 1)
    logits = jnp.zeros((t, k_count), jnp.float32)
    for k in range(k_count):
        row = gp_ref[k]
        u = lax.bitcast_convert_type(row[:, 0:64], jnp.uint32)
        s_lo = _unpack_lo(u)
        s_hi = _unpack_hi(u)
        nc = row[:, 88:91]
        diff = cc - nc
        dist = jnp.sqrt(jnp.sum(diff * diff, axis=1, keepdims=True) + 1e-6)
        h = (cp + jnp.dot(s_lo, w1n_lo, preferred_element_type=jnp.float32)
             + jnp.dot(s_hi, w1n_hi, preferred_element_type=jnp.float32)
             + dist * w1d)
        h = _gelu_exact(h)
        logit = jnp.sum(h * w2, axis=1, keepdims=True) + b2_ref[:, 0:1]
        logit = jnp.clip(logit, -10000.0, 10.0)
        logits = jnp.where(lane == k, logit, logits)

    m = jnp.max(logits, axis=1, keepdims=True)
    e = jnp.exp(logits - m)
    att = e / jnp.sum(e, axis=1, keepdims=True)

    smsg_lo = jnp.zeros((t, 64), jnp.float32)
    smsg_hi = jnp.zeros((t, 64), jnp.float32)
    vmsg_lo = jnp.zeros((t, 24), jnp.float32)
    vmsg_hi = jnp.zeros((t, 24), jnp.float32)
    for k in range(k_count):
        row = gp_ref[k]
        a = att[:, k:k + 1]
        us = lax.bitcast_convert_type(row[:, 0:64], jnp.uint32)
        uv = lax.bitcast_convert_type(row[:, 64:88], jnp.uint32)
        smsg_lo = smsg_lo + a * _unpack_lo(us)
        smsg_hi = smsg_hi + a * _unpack_hi(us)
        vmsg_lo = vmsg_lo + a * _unpack_lo(uv)
        vmsg_hi = vmsg_hi + a * _unpack_hi(uv)

    smsg = jnp.concatenate([smsg_lo, smsg_hi], axis=1)
    x = tab_ref[...] + smsg
    mu = jnp.mean(x, axis=1, keepdims=True)
    xc = x - mu
    var = jnp.mean(xc * xc, axis=1, keepdims=True)
    x = xc * jax.lax.rsqrt(var + 1e-5) * lng_ref[...] + lnb_ref[...]

    f = jnp.dot(x, wf1_ref[...], preferred_element_type=jnp.float32) + bf1_ref[...]
    f = _gelu_exact(f)
    f = jnp.dot(f, wf2_ref[...], preferred_element_type=jnp.float32) + bf2_ref[...]

    outs_ref[...] = x + f
    outv_ref[...] = jnp.concatenate([vmsg_lo, vmsg_hi], axis=1)


def _main_call(gp3, table_s, cproj, coords, w1n, w1d, w2, b2, wf1, bf1,
               wf2, bf2, lng, lnb, tile, k_count):
    n = table_s.shape[0]
    grid = n // tile
    const = lambda i: (0, 0)
    return pl.pallas_call(
        functools.partial(_main_body, k_count=k_count),
        grid=(grid,),
        in_specs=[
            pl.BlockSpec((k_count, tile, PW), lambda i: (0, i, 0)),
            pl.BlockSpec((tile, CS), lambda i: (i, 0)),
            pl.BlockSpec((tile, CPW), lambda i: (i, 0)),
            pl.BlockSpec((tile, 3), lambda i: (i, 0)),
            pl.BlockSpec((CS, CS), const),
            pl.BlockSpec((1, CS), const),
            pl.BlockSpec((1, CS), const),
            pl.BlockSpec((1, CS), const),
            pl.BlockSpec((CS, 4 * CS), const),
            pl.BlockSpec((1, 4 * CS), const),
            pl.BlockSpec((4 * CS, CS), const),
            pl.BlockSpec((1, CS), const),
            pl.BlockSpec((1, CS), const),
            pl.BlockSpec((1, CS), const),
        ],
        out_specs=[
            pl.BlockSpec((tile, CS), lambda i: (i, 0)),
            pl.BlockSpec((tile, VD * 3), lambda i: (i, 0)),
        ],
        out_shape=[
            jax.ShapeDtypeStruct((n, CS), jnp.float32),
            jax.ShapeDtypeStruct((n, VD * 3), jnp.float32),
        ],
    )(gp3, table_s, cproj, coords, w1n, w1d, w2, b2, wf1, bf1, wf2, bf2,
      lng, lnb)


def kernel(scalar_feats, coords, E_idx, W_e3, b_e3, W_att1, b_att1, W_att2,
           b_att2, W_ffn1, b_ffn1, W_ffn2, b_ffn2, ln_g, ln_b):
    b, n, cs = scalar_feats.shape
    k_count = E_idx.shape[-1]
    sf = scalar_feats[0]
    co = coords[0]

    w1c = W_att1[:, 0:cs]
    w1n = W_att1[:, cs:2 * cs]
    w1d = W_att1[:, 2 * cs]
    we3s = W_e3[0:cs, :]
    wcat = jnp.concatenate([W_e3.T, (w1c @ we3s).T], axis=1)
    bcat = jnp.concatenate([b_e3, w1c @ b_e3[0:cs] + b_att1])[None, :]

    table_s, table_p, cproj = _prepass_call(sf, co, wcat, bcat, tile=1000)

    idx = jnp.transpose(E_idx[0]).reshape(-1).astype(jnp.int32)
    g_p = _make_gather(n * k_count)(table_p, idx)
    gp3 = g_p.reshape(k_count, n, PW)

    outs, outv = _main_call(
        gp3, table_s, cproj, co,
        w1n.T, w1d[None, :], W_att2[0][None, :],
        jnp.broadcast_to(b_att2.reshape(1, 1), (1, CS)),
        W_ffn1.T, b_ffn1[None, :], W_ffn2.T, b_ffn2[None, :],
        ln_g[None, :], ln_b[None, :],
        tile=200, k_count=k_count,
    )
    return outs[None], outv.reshape(1, n, VD, 3)


# speedup vs baseline: 4.7402x; 1.2114x over previous
import functools

import jax
import jax.numpy as jnp
from jax import lax
from jax.experimental import pallas as pl
from jax.experimental.pallas import tpu as pltpu
from jax.experimental.pallas import tpu_sc as plsc

CS = 128
VD = 16
PW = 128
CPW = 128


def _gelu_exact(x):
    return 0.5 * x * (1.0 + lax.erf(x * 0.7071067811865476))


def _f32_to_f16bits(x):
    u = lax.bitcast_convert_type(x, jnp.uint32)
    s = (u >> jnp.uint32(16)) & jnp.uint32(0x8000)
    mag = u & jnp.uint32(0x7FFFFFFF)
    core = (mag - jnp.uint32(0x38000000) + jnp.uint32(0x1000)) >> jnp.uint32(13)
    core = jnp.where(mag < jnp.uint32(0x38800000), jnp.uint32(0), core)
    core = jnp.where(mag >= jnp.uint32(0x47800000), jnp.uint32(0x7BFF), core)
    return s | core


def _f16bits_to_f32(h):
    s = (h & jnp.uint32(0x8000)) << jnp.uint32(16)
    core = h & jnp.uint32(0x7FFF)
    mag = (core << jnp.uint32(13)) + jnp.uint32(0x38000000)
    mag = jnp.where(core == jnp.uint32(0), jnp.uint32(0), mag)
    return lax.bitcast_convert_type(s | mag, jnp.float32)


def _pack16(a, b):
    w = _f32_to_f16bits(a) | (_f32_to_f16bits(b) << jnp.uint32(16))
    return lax.bitcast_convert_type(w, jnp.float32)


def _unpack_lo(u):
    return _f16bits_to_f32(u & jnp.uint32(0xFFFF))


def _unpack_hi(u):
    return _f16bits_to_f32(u >> jnp.uint32(16))


def _prepass_body(sf_ref, co_ref, wcat_ref, bcat_ref, ts_ref, tp_ref, cproj_ref):
    lin = jnp.dot(sf_ref[...], wcat_ref[...],
                  preferred_element_type=jnp.float32) + bcat_ref[...]
    t = sf_ref.shape[0]
    s = lin[:, 0:CS]
    v = lin[:, CS:176]
    ts_ref[...] = s
    cproj_ref[...] = lin[:, 176:304]
    tp_ref[...] = jnp.concatenate(
        [_pack16(s[:, 0:64], s[:, 64:128]),
         _pack16(v[:, 0:24], v[:, 24:48]),
         co_ref[...],
         jnp.zeros((t, PW - 91), jnp.float32)], axis=1)


def _prepass_call(sf, coords, wcat, bcat, tile):
    n = sf.shape[0]
    grid = n // tile
    return pl.pallas_call(
        _prepass_body,
        grid=(grid,),
        in_specs=[
            pl.BlockSpec((tile, CS), lambda i: (i, 0)),
            pl.BlockSpec((tile, 3), lambda i: (i, 0)),
            pl.BlockSpec((CS, 304), lambda i: (0, 0)),
            pl.BlockSpec((1, 304), lambda i: (0, 0)),
        ],
        out_specs=[
            pl.BlockSpec((tile, CS), lambda i: (i, 0)),
            pl.BlockSpec((tile, PW), lambda i: (i, 0)),
            pl.BlockSpec((tile, CPW), lambda i: (i, 0)),
        ],
        out_shape=[
            jax.ShapeDtypeStruct((n, CS), jnp.float32),
            jax.ShapeDtypeStruct((n, PW), jnp.float32),
            jax.ShapeDtypeStruct((n, CPW), jnp.float32),
        ],
    )(sf, coords, wcat, bcat)


_NC = 2
_NS = 16
_NW = _NC * _NS
_CHUNK = 80
_NBUF = 5


def _make_gather(ne):
    per_w = ne // _NW
    n_chunks = per_w // _CHUNK
    mesh = plsc.VectorSubcoreMesh(core_axis_name="c", subcore_axis_name="s")

    @functools.partial(
        pl.kernel, mesh=mesh,
        out_type=jax.ShapeDtypeStruct((ne, PW), jnp.float32),
        scratch_types=[
            pltpu.VMEM((n_chunks, _CHUNK), jnp.int32),
            pltpu.VMEM((_NBUF, _CHUNK, PW), jnp.float32),
        ] + [pltpu.SemaphoreType.DMA] * (2 * _NBUF),
    )
    def gather(tp_hbm, idx_hbm, outp_hbm, idx_v, bufs, *sems):
        gsem = sems[:_NBUF]
        ssem = sems[_NBUF:]
        wid = lax.axis_index("s") * _NC + lax.axis_index("c")
        base = wid * per_w

        pltpu.sync_copy(idx_hbm.at[wid], idx_v)

        pend_g = [None] * _NBUF
        pend_s = [None] * _NBUF
        for c in range(n_chunks + _NBUF - 1):
            if c < n_chunks:
                b = c % _NBUF
                if pend_s[b] is not None:
                    pend_s[b].wait()
                pend_g[b] = pltpu.async_copy(
                    tp_hbm.at[idx_v.at[c]], bufs.at[b], gsem[b])
            d = c - (_NBUF - 1)
            if d >= 0:
                bd = d % _NBUF
                pend_g[bd].wait()
                off = pl.multiple_of(base + d * _CHUNK, 8)
                pend_s[bd] = pltpu.async_copy(
                    bufs.at[bd], outp_hbm.at[pl.ds(off, _CHUNK)], ssem[bd])
        for b in range(_NBUF):
            if pend_s[b] is not None:
                pend_s[b].wait()

    return gather


def _main_body(gp_ref, tab_ref, cp_ref, co_ref, w1n_ref, w1d_ref,
               w2_ref, b2_ref, wf1_ref, bf1_ref, wf2_ref, bf2_ref, lng_ref,
               lnb_ref, outs_ref, outv_ref, k_count):
    t = tab_ref.shape[0]
    cc = co_ref[...]
    cp = cp_ref[...]
    w1n_lo = w1n_ref[0:64, :]
    w1n_hi = w1n_ref[64:CS, :]
    w1d = w1d_ref[...]
    w2 = w2_ref[...]

    lane = lax.broadcasted_iota(jnp.int32, (t, k_count), 1)
    logits = jnp.zeros((t, k_count), jnp.float32)
    for k in range(k_count):
        row = gp_ref[k]
        u = lax.bitcast_convert_type(row[:, 0:64], jnp.uint32)
        s_lo = _unpack_lo(u)
        s_hi = _unpack_hi(u)
        nc = row[:, 88:91]
        diff = cc - nc
        dist = jnp.sqrt(jnp.sum(diff * diff, axis=1, keepdims=True) + 1e-6)
        h = (cp + jnp.dot(s_lo, w1n_lo, preferred_element_type=jnp.float32)
             + jnp.dot(s_hi, w1n_hi, preferred_element_type=jnp.float32)
             + dist * w1d)
        h = _gelu_exact(h)
        logit = jnp.sum(h * w2, axis=1, keepdims=True) + b2_ref[:, 0:1]
        logit = jnp.clip(logit, -10000.0, 10.0)
        logits = jnp.where(lane == k, logit, logits)

    m = jnp.max(logits, axis=1, keepdims=True)
    e = jnp.exp(logits - m)
    att = e / jnp.sum(e, axis=1, keepdims=True)

    smsg_lo = jnp.zeros((t, 64), jnp.float32)
    smsg_hi = jnp.zeros((t, 64), jnp.float32)
    vmsg_lo = jnp.zeros((t, 24), jnp.float32)
    vmsg_hi = jnp.zeros((t, 24), jnp.float32)
    for k in range(k_count):
        row = gp_ref[k]
        a = att[:, k:k + 1]
        us = lax.bitcast_convert_type(row[:, 0:64], jnp.uint32)
        uv = lax.bitcast_convert_type(row[:, 64:88], jnp.uint32)
        smsg_lo = smsg_lo + a * _unpack_lo(us)
        smsg_hi = smsg_hi + a * _unpack_hi(us)
        vmsg_lo = vmsg_lo + a * _unpack_lo(uv)
        vmsg_hi = vmsg_hi + a * _unpack_hi(uv)

    smsg = jnp.concatenate([smsg_lo, smsg_hi], axis=1)
    x = tab_ref[...] + smsg
    mu = jnp.mean(x, axis=1, keepdims=True)
    xc = x - mu
    var = jnp.mean(xc * xc, axis=1, keepdims=True)
    x = xc * jax.lax.rsqrt(var + 1e-5) * lng_ref[...] + lnb_ref[...]

    f = jnp.dot(x, wf1_ref[...], preferred_element_type=jnp.float32) + bf1_ref[...]
    f = _gelu_exact(f)
    f = jnp.dot(f, wf2_ref[...], preferred_element_type=jnp.float32) + bf2_ref[...]

    outs_ref[...] = x + f
    outv_ref[...] = jnp.concatenate([vmsg_lo, vmsg_hi], axis=1)


def _main_call(gp3, table_s, cproj, coords, w1n, w1d, w2, b2, wf1, bf1,
               wf2, bf2, lng, lnb, tile, k_count):
    n = table_s.shape[0]
    grid = n // tile
    const = lambda i: (0, 0)
    return pl.pallas_call(
        functools.partial(_main_body, k_count=k_count),
        grid=(grid,),
        in_specs=[
            pl.BlockSpec((k_count, tile, PW), lambda i: (0, i, 0)),
            pl.BlockSpec((tile, CS), lambda i: (i, 0)),
            pl.BlockSpec((tile, CPW), lambda i: (i, 0)),
            pl.BlockSpec((tile, 3), lambda i: (i, 0)),
            pl.BlockSpec((CS, CS), const),
            pl.BlockSpec((1, CS), const),
            pl.BlockSpec((1, CS), const),
            pl.BlockSpec((1, CS), const),
            pl.BlockSpec((CS, 4 * CS), const),
            pl.BlockSpec((1, 4 * CS), const),
            pl.BlockSpec((4 * CS, CS), const),
            pl.BlockSpec((1, CS), const),
            pl.BlockSpec((1, CS), const),
            pl.BlockSpec((1, CS), const),
        ],
        out_specs=[
            pl.BlockSpec((tile, CS), lambda i: (i, 0)),
            pl.BlockSpec((tile, VD * 3), lambda i: (i, 0)),
        ],
        out_shape=[
            jax.ShapeDtypeStruct((n, CS), jnp.float32),
            jax.ShapeDtypeStruct((n, VD * 3), jnp.float32),
        ],
    )(gp3, table_s, cproj, coords, w1n, w1d, w2, b2, wf1, bf1, wf2, bf2,
      lng, lnb)


def kernel(scalar_feats, coords, E_idx, W_e3, b_e3, W_att1, b_att1, W_att2,
           b_att2, W_ffn1, b_ffn1, W_ffn2, b_ffn2, ln_g, ln_b):
    b, n, cs = scalar_feats.shape
    k_count = E_idx.shape[-1]
    sf = scalar_feats[0]
    co = coords[0]

    w1c = W_att1[:, 0:cs]
    w1n = W_att1[:, cs:2 * cs]
    w1d = W_att1[:, 2 * cs]
    we3s = W_e3[0:cs, :]
    wcat = jnp.concatenate([W_e3.T, (w1c @ we3s).T], axis=1)
    bcat = jnp.concatenate([b_e3, w1c @ b_e3[0:cs] + b_att1])[None, :]

    table_s, table_p, cproj = _prepass_call(sf, co, wcat, bcat, tile=1000)

    ne = n * k_count
    per_w = ne // _NW
    idx = (jnp.transpose(E_idx[0]).reshape(-1).astype(jnp.int32)
           .reshape(_NW, per_w // _CHUNK, _CHUNK))
    g_p = _make_gather(ne)(table_p, idx)
    gp3 = g_p.reshape(k_count, n, PW)

    outs, outv = _main_call(
        gp3, table_s, cproj, co,
        w1n.T, w1d[None, :], W_att2[0][None, :],
        jnp.broadcast_to(b_att2.reshape(1, 1), (1, CS)),
        W_ffn1.T, b_ffn1[None, :], W_ffn2.T, b_ffn2[None, :],
        ln_g[None, :], ln_b[None, :],
        tile=200, k_count=k_count,
    )
    return outs[None], outv.reshape(1, n, VD, 3)


# speedup vs baseline: 5.2040x; 1.0979x over previous
import functools

import jax
import jax.numpy as jnp
from jax import lax
from jax.experimental import pallas as pl
from jax.experimental.pallas import tpu as pltpu
from jax.experimental.pallas import tpu_sc as plsc

CS = 128
VD = 16
PW = 128
CPW = 128


def _gelu_exact(x):
    return 0.5 * x * (1.0 + lax.erf(x * 0.7071067811865476))


def _pack16(a, b):
    ua = (lax.bitcast_convert_type(a, jnp.uint32) + jnp.uint32(0x8000)) >> jnp.uint32(16)
    ub = (lax.bitcast_convert_type(b, jnp.uint32) + jnp.uint32(0x8000)) & jnp.uint32(0xFFFF0000)
    return lax.bitcast_convert_type(ua | ub, jnp.float32)


def _unpack_lo(u):
    return lax.bitcast_convert_type(u << jnp.uint32(16), jnp.float32)


def _unpack_hi(u):
    return lax.bitcast_convert_type(u & jnp.uint32(0xFFFF0000), jnp.float32)


def _prepass_body(sf_ref, co_ref, wcat_ref, bcat_ref, ts_ref, tp_ref, cproj_ref):
    lin = jnp.dot(sf_ref[...], wcat_ref[...],
                  preferred_element_type=jnp.float32) + bcat_ref[...]
    t = sf_ref.shape[0]
    s = lin[:, 0:CS]
    v = lin[:, CS:176]
    ts_ref[...] = s
    cproj_ref[...] = lin[:, 176:304]
    co = co_ref[...]
    cj2 = jnp.sum(co * co, axis=1, keepdims=True)
    tp_ref[...] = jnp.concatenate(
        [_pack16(s[:, 0:64], s[:, 64:128]),
         _pack16(v[:, 0:24], v[:, 24:48]),
         co, cj2,
         jnp.zeros((t, PW - 92), jnp.float32)], axis=1)


def _prepass_call(sf, coords, wcat, bcat, tile):
    n = sf.shape[0]
    grid = n // tile
    return pl.pallas_call(
        _prepass_body,
        grid=(grid,),
        in_specs=[
            pl.BlockSpec((tile, CS), lambda i: (i, 0)),
            pl.BlockSpec((tile, 3), lambda i: (i, 0)),
            pl.BlockSpec((CS, 304), lambda i: (0, 0)),
            pl.BlockSpec((1, 304), lambda i: (0, 0)),
        ],
        out_specs=[
            pl.BlockSpec((tile, CS), lambda i: (i, 0)),
            pl.BlockSpec((tile, PW), lambda i: (i, 0)),
            pl.BlockSpec((tile, CPW), lambda i: (i, 0)),
        ],
        out_shape=[
            jax.ShapeDtypeStruct((n, CS), jnp.float32),
            jax.ShapeDtypeStruct((n, PW), jnp.float32),
            jax.ShapeDtypeStruct((n, CPW), jnp.float32),
        ],
    )(sf, coords, wcat, bcat)


_NC = 2
_NS = 16
_NW = _NC * _NS
_NBUF = 5


def _make_gather(ne, chunk):
    per_w = ne // _NW
    n_chunks = per_w // chunk
    mesh = plsc.VectorSubcoreMesh(core_axis_name="c", subcore_axis_name="s")

    @functools.partial(
        pl.kernel, mesh=mesh,
        out_type=jax.ShapeDtypeStruct((ne, PW), jnp.float32),
        scratch_types=[
            pltpu.VMEM((n_chunks, chunk), jnp.int32),
            pltpu.VMEM((_NBUF, chunk, PW), jnp.float32),
        ] + [pltpu.SemaphoreType.DMA] * (2 * _NBUF),
    )
    def gather(tp_hbm, idx_hbm, outp_hbm, idx_v, bufs, *sems):
        gsem = sems[:_NBUF]
        ssem = sems[_NBUF:]
        wid = lax.axis_index("s") * _NC + lax.axis_index("c")
        base = wid * per_w

        pltpu.sync_copy(idx_hbm.at[wid], idx_v)

        pend_g = [None] * _NBUF
        pend_s = [None] * _NBUF
        for c in range(n_chunks + _NBUF - 1):
            if c < n_chunks:
                b = c % _NBUF
                if pend_s[b] is not None:
                    pend_s[b].wait()
                pend_g[b] = pltpu.async_copy(
                    tp_hbm.at[idx_v.at[c]], bufs.at[b], gsem[b])
            d = c - (_NBUF - 1)
            if d >= 0:
                bd = d % _NBUF
                pend_g[bd].wait()
                off = pl.multiple_of(base + d * chunk, 8)
                pend_s[bd] = pltpu.async_copy(
                    bufs.at[bd], outp_hbm.at[pl.ds(off, chunk)], ssem[bd])
        for b in range(_NBUF):
            if pend_s[b] is not None:
                pend_s[b].wait()

    return gather


def _main_body(gp_ref, tab_ref, cp_ref, co_ref, w1n_ref, w1d_ref,
               w2m_ref, b2_ref, wf1_ref, bf1_ref, wf2_ref, bf2_ref, lng_ref,
               lnb_ref, outs_ref, outv_ref, k_count):
    t = tab_ref.shape[0]
    cc = co_ref[...]
    cp = cp_ref[...]
    w1n_lo = w1n_ref[0:64, :]
    w1n_hi = w1n_ref[64:CS, :]
    w1d = w1d_ref[...]

    ccx = cc[:, 0:1]
    ccy = cc[:, 1:2]
    ccz = cc[:, 2:3]
    ci2 = ccx * ccx + ccy * ccy + ccz * ccz

    lacc = jnp.zeros((t, CS), jnp.float32)
    for k in range(k_count):
        row = gp_ref[k]
        u = lax.bitcast_convert_type(row[:, 0:64], jnp.uint32)
        s_lo = _unpack_lo(u)
        s_hi = _unpack_hi(u)
        dot3 = ccx * row[:, 88:89] + ccy * row[:, 89:90] + ccz * row[:, 90:91]
        d2 = ci2 + row[:, 91:92] - 2.0 * dot3
        dist = jnp.sqrt(jnp.maximum(d2, 0.0) + 1e-6)
        h = (cp + jnp.dot(s_lo, w1n_lo, preferred_element_type=jnp.float32)
             + jnp.dot(s_hi, w1n_hi, preferred_element_type=jnp.float32)
             + dist * w1d)
        h = _gelu_exact(h)
        lacc = lacc + jnp.dot(h, w2m_ref[k], preferred_element_type=jnp.float32)

    logits = jnp.clip(lacc[:, 0:k_count] + b2_ref[:, 0:k_count], -10000.0, 10.0)
    m = jnp.max(logits, axis=1, keepdims=True)
    e = jnp.exp(logits - m)
    att = e / jnp.sum(e, axis=1, keepdims=True)

    smsg_lo = jnp.zeros((t, 64), jnp.float32)
    smsg_hi = jnp.zeros((t, 64), jnp.float32)
    vmsg_lo = jnp.zeros((t, 24), jnp.float32)
    vmsg_hi = jnp.zeros((t, 24), jnp.float32)
    for k in range(k_count):
        row = gp_ref[k]
        a = att[:, k:k + 1]
        us = lax.bitcast_convert_type(row[:, 0:64], jnp.uint32)
        uv = lax.bitcast_convert_type(row[:, 64:88], jnp.uint32)
        smsg_lo = smsg_lo + a * _unpack_lo(us)
        smsg_hi = smsg_hi + a * _unpack_hi(us)
        vmsg_lo = vmsg_lo + a * _unpack_lo(uv)
        vmsg_hi = vmsg_hi + a * _unpack_hi(uv)

    smsg = jnp.concatenate([smsg_lo, smsg_hi], axis=1)
    x = tab_ref[...] + smsg
    mu = jnp.mean(x, axis=1, keepdims=True)
    xc = x - mu
    var = jnp.mean(xc * xc, axis=1, keepdims=True)
    x = xc * jax.lax.rsqrt(var + 1e-5) * lng_ref[...] + lnb_ref[...]

    f = jnp.dot(x, wf1_ref[...], preferred_element_type=jnp.float32) + bf1_ref[...]
    f = _gelu_exact(f)
    f = jnp.dot(f, wf2_ref[...], preferred_element_type=jnp.float32) + bf2_ref[...]

    outs_ref[...] = x + f
    outv_ref[...] = jnp.concatenate([vmsg_lo, vmsg_hi], axis=1)


def _main_call(gp3, table_s, cproj, coords, w1n, w1d, w2m, b2, wf1, bf1,
               wf2, bf2, lng, lnb, tile, k_count):
    n = table_s.shape[0]
    grid = n // tile
    const = lambda i: (0, 0)
    return pl.pallas_call(
        functools.partial(_main_body, k_count=k_count),
        grid=(grid,),
        in_specs=[
            pl.BlockSpec((k_count, tile, PW), lambda i: (0, i, 0)),
            pl.BlockSpec((tile, CS), lambda i: (i, 0)),
            pl.BlockSpec((tile, CPW), lambda i: (i, 0)),
            pl.BlockSpec((tile, 3), lambda i: (i, 0)),
            pl.BlockSpec((CS, CS), const),
            pl.BlockSpec((1, CS), const),
            pl.BlockSpec((k_count, CS, CS), lambda i: (0, 0, 0)),
            pl.BlockSpec((1, CS), const),
            pl.BlockSpec((CS, 4 * CS), const),
            pl.BlockSpec((1, 4 * CS), const),
            pl.BlockSpec((4 * CS, CS), const),
            pl.BlockSpec((1, CS), const),
            pl.BlockSpec((1, CS), const),
            pl.BlockSpec((1, CS), const),
        ],
        out_specs=[
            pl.BlockSpec((tile, CS), lambda i: (i, 0)),
            pl.BlockSpec((tile, VD * 3), lambda i: (i, 0)),
        ],
        out_shape=[
            jax.ShapeDtypeStruct((n, CS), jnp.float32),
            jax.ShapeDtypeStruct((n, VD * 3), jnp.float32),
        ],
    )(gp3, table_s, cproj, coords, w1n, w1d, w2m, b2, wf1, bf1, wf2, bf2,
      lng, lnb)


def kernel(scalar_feats, coords, E_idx, W_e3, b_e3, W_att1, b_att1, W_att2,
           b_att2, W_ffn1, b_ffn1, W_ffn2, b_ffn2, ln_g, ln_b):
    b, n, cs = scalar_feats.shape
    k_count = E_idx.shape[-1]
    sf = scalar_feats[0]
    co = coords[0]

    w1c = W_att1[:, 0:cs]
    w1n = W_att1[:, cs:2 * cs]
    w1d = W_att1[:, 2 * cs]
    we3s = W_e3[0:cs, :]
    wcat = jnp.concatenate([W_e3.T, (w1c @ we3s).T], axis=1)
    bcat = jnp.concatenate([b_e3, w1c @ b_e3[0:cs] + b_att1])[None, :]

    table_s, table_p, cproj = _prepass_call(sf, co, wcat, bcat, tile=1000)

    onehot = (jnp.arange(k_count)[:, None] ==
              jnp.arange(CS)[None, :]).astype(jnp.float32)
    w2m = W_att2[0][None, :, None] * onehot[:, None, :]
    b2 = jnp.broadcast_to(b_att2.reshape(1, 1), (1, CS))

    nh = n // 2
    ne_h = nh * k_count
    chunk = 40
    ei = E_idx[0].astype(jnp.int32)
    idx_a = (jnp.transpose(ei[0:nh]).reshape(-1)
             .reshape(_NW, (ne_h // _NW) // chunk, chunk))
    idx_b = (jnp.transpose(ei[nh:n]).reshape(-1)
             .reshape(_NW, (ne_h // _NW) // chunk, chunk))
    g_a = _make_gather(ne_h, chunk)(table_p, idx_a)
    g_b = _make_gather(ne_h, chunk)(table_p, idx_b)

    halves = []
    for g_h, sl in ((g_a, slice(0, nh)), (g_b, slice(nh, n))):
        halves.append(_main_call(
            g_h.reshape(k_count, nh, PW), table_s[sl], cproj[sl], co[sl],
            w1n.T, w1d[None, :], w2m, b2,
            W_ffn1.T, b_ffn1[None, :], W_ffn2.T, b_ffn2[None, :],
            ln_g[None, :], ln_b[None, :],
            tile=200, k_count=k_count,
        ))
    outs = jnp.concatenate([halves[0][0], halves[1][0]], axis=0)
    outv = jnp.concatenate([halves[0][1], halves[1][1]], axis=0)
    return outs[None], outv.reshape(1, n, VD, 3)


# speedup vs baseline: 5.2662x; 1.0119x over previous
import functools

import jax
import jax.numpy as jnp
from jax import lax
from jax.experimental import pallas as pl
from jax.experimental.pallas import tpu as pltpu
from jax.experimental.pallas import tpu_sc as plsc

CS = 128
VD = 16
PW = 128
CPW = 128


def _gelu_exact(x):
    return 0.5 * x * (1.0 + lax.erf(x * 0.7071067811865476))


def _pack16(a, b):
    ua = (lax.bitcast_convert_type(a, jnp.uint32) + jnp.uint32(0x8000)) >> jnp.uint32(16)
    ub = (lax.bitcast_convert_type(b, jnp.uint32) + jnp.uint32(0x8000)) & jnp.uint32(0xFFFF0000)
    return lax.bitcast_convert_type(ua | ub, jnp.float32)


def _unpack_lo(u):
    return lax.bitcast_convert_type(u << jnp.uint32(16), jnp.float32)


def _unpack_hi(u):
    return lax.bitcast_convert_type(u & jnp.uint32(0xFFFF0000), jnp.float32)


def _prepass_body(sf_ref, co_ref, wcat_ref, bcat_ref, ts_ref, tp_ref, cproj_ref):
    lin = jnp.dot(sf_ref[...], wcat_ref[...],
                  preferred_element_type=jnp.float32) + bcat_ref[...]
    t = sf_ref.shape[0]
    s = lin[:, 0:CS]
    v = lin[:, CS:176]
    ts_ref[...] = s
    cproj_ref[...] = lin[:, 176:304]
    co = co_ref[...]
    cj2 = jnp.sum(co * co, axis=1, keepdims=True)
    tp_ref[...] = jnp.concatenate(
        [_pack16(s[:, 0:64], s[:, 64:128]),
         _pack16(v[:, 0:24], v[:, 24:48]),
         co, cj2,
         jnp.zeros((t, PW - 92), jnp.float32)], axis=1)


def _prepass_call(sf, coords, wcat, bcat, tile):
    n = sf.shape[0]
    grid = n // tile
    return pl.pallas_call(
        _prepass_body,
        grid=(grid,),
        in_specs=[
            pl.BlockSpec((tile, CS), lambda i: (i, 0)),
            pl.BlockSpec((tile, 3), lambda i: (i, 0)),
            pl.BlockSpec((CS, 304), lambda i: (0, 0)),
            pl.BlockSpec((1, 304), lambda i: (0, 0)),
        ],
        out_specs=[
            pl.BlockSpec((tile, CS), lambda i: (i, 0)),
            pl.BlockSpec((tile, PW), lambda i: (i, 0)),
            pl.BlockSpec((tile, CPW), lambda i: (i, 0)),
        ],
        out_shape=[
            jax.ShapeDtypeStruct((n, CS), jnp.float32),
            jax.ShapeDtypeStruct((n, PW), jnp.float32),
            jax.ShapeDtypeStruct((n, CPW), jnp.float32),
        ],
    )(sf, coords, wcat, bcat)


_NC = 2
_NS = 16
_NW = _NC * _NS
_NBUF = 5


def _make_gather(ne, chunk):
    per_w = ne // _NW
    n_chunks = per_w // chunk
    mesh = plsc.VectorSubcoreMesh(core_axis_name="c", subcore_axis_name="s")

    @functools.partial(
        pl.kernel, mesh=mesh,
        out_type=jax.ShapeDtypeStruct((ne, PW), jnp.float32),
        scratch_types=[
            pltpu.VMEM((n_chunks, chunk), jnp.int32),
            pltpu.VMEM((_NBUF, chunk, PW), jnp.float32),
        ] + [pltpu.SemaphoreType.DMA] * (2 * _NBUF),
    )
    def gather(tp_hbm, idx_hbm, outp_hbm, idx_v, bufs, *sems):
        gsem = sems[:_NBUF]
        ssem = sems[_NBUF:]
        wid = lax.axis_index("s") * _NC + lax.axis_index("c")
        base = wid * per_w

        pltpu.sync_copy(idx_hbm.at[wid], idx_v)

        pend_g = [None] * _NBUF
        pend_s = [None] * _NBUF
        for c in range(n_chunks + _NBUF - 1):
            if c < n_chunks:
                b = c % _NBUF
                if pend_s[b] is not None:
                    pend_s[b].wait()
                pend_g[b] = pltpu.async_copy(
                    tp_hbm.at[idx_v.at[c]], bufs.at[b], gsem[b])
            d = c - (_NBUF - 1)
            if d >= 0:
                bd = d % _NBUF
                pend_g[bd].wait()
                off = pl.multiple_of(base + d * chunk, 8)
                pend_s[bd] = pltpu.async_copy(
                    bufs.at[bd], outp_hbm.at[pl.ds(off, chunk)], ssem[bd])
        for b in range(_NBUF):
            if pend_s[b] is not None:
                pend_s[b].wait()

    return gather


def _main_body(gp_ref, tab_ref, cp_ref, co_ref, w1n_ref, w1d_ref,
               w2m_ref, b2_ref, wf1_ref, bf1_ref, wf2_ref, bf2_ref, lng_ref,
               lnb_ref, outs_ref, outv_ref, k_count):
    t = tab_ref.shape[0]
    cc = co_ref[...]
    cp = cp_ref[...]
    w1n_lo = w1n_ref[0:64, :]
    w1n_hi = w1n_ref[64:CS, :]
    w1d = w1d_ref[...]

    ccx = cc[:, 0:1]
    ccy = cc[:, 1:2]
    ccz = cc[:, 2:3]
    ci2 = ccx * ccx + ccy * ccy + ccz * ccz

    lacc = jnp.zeros((t, CS), jnp.float32)
    for k in range(k_count):
        row = gp_ref[k]
        u = lax.bitcast_convert_type(row[:, 0:64], jnp.uint32)
        s_lo = _unpack_lo(u)
        s_hi = _unpack_hi(u)
        dot3 = ccx * row[:, 88:89] + ccy * row[:, 89:90] + ccz * row[:, 90:91]
        d2 = ci2 + row[:, 91:92] - 2.0 * dot3
        dist = jnp.sqrt(jnp.maximum(d2, 0.0) + 1e-6)
        h = (cp + jnp.dot(s_lo, w1n_lo, preferred_element_type=jnp.float32)
             + jnp.dot(s_hi, w1n_hi, preferred_element_type=jnp.float32)
             + dist * w1d)
        h = _gelu_exact(h)
        lacc = lacc + jnp.dot(h, w2m_ref[k], preferred_element_type=jnp.float32)

    logits = jnp.clip(lacc[:, 0:k_count] + b2_ref[:, 0:k_count], -10000.0, 10.0)
    m = jnp.max(logits, axis=1, keepdims=True)
    e = jnp.exp(logits - m)
    att = e / jnp.sum(e, axis=1, keepdims=True)

    smsg_lo = jnp.zeros((t, 64), jnp.float32)
    smsg_hi = jnp.zeros((t, 64), jnp.float32)
    vmsg_lo = jnp.zeros((t, 24), jnp.float32)
    vmsg_hi = jnp.zeros((t, 24), jnp.float32)
    for k in range(k_count):
        row = gp_ref[k]
        a = att[:, k:k + 1]
        us = lax.bitcast_convert_type(row[:, 0:64], jnp.uint32)
        uv = lax.bitcast_convert_type(row[:, 64:88], jnp.uint32)
        smsg_lo = smsg_lo + a * _unpack_lo(us)
        smsg_hi = smsg_hi + a * _unpack_hi(us)
        vmsg_lo = vmsg_lo + a * _unpack_lo(uv)
        vmsg_hi = vmsg_hi + a * _unpack_hi(uv)

    smsg = jnp.concatenate([smsg_lo, smsg_hi], axis=1)
    x = tab_ref[...] + smsg
    mu = jnp.mean(x, axis=1, keepdims=True)
    xc = x - mu
    var = jnp.mean(xc * xc, axis=1, keepdims=True)
    x = xc * jax.lax.rsqrt(var + 1e-5) * lng_ref[...] + lnb_ref[...]

    f = jnp.dot(x, wf1_ref[...], preferred_element_type=jnp.float32) + bf1_ref[...]
    f = _gelu_exact(f)
    f = jnp.dot(f, wf2_ref[...], preferred_element_type=jnp.float32) + bf2_ref[...]

    outs_ref[...] = x + f
    outv_ref[...] = jnp.concatenate([vmsg_lo, vmsg_hi], axis=1)


def _main_call(gp3, table_s, cproj, coords, w1n, w1d, w2m, b2, wf1, bf1,
               wf2, bf2, lng, lnb, tile, k_count):
    n = table_s.shape[0]
    grid = n // tile
    const = lambda i: (0, 0)
    return pl.pallas_call(
        functools.partial(_main_body, k_count=k_count),
        grid=(grid,),
        in_specs=[
            pl.BlockSpec((k_count, tile, PW), lambda i: (0, i, 0)),
            pl.BlockSpec((tile, CS), lambda i: (i, 0)),
            pl.BlockSpec((tile, CPW), lambda i: (i, 0)),
            pl.BlockSpec((tile, 3), lambda i: (i, 0)),
            pl.BlockSpec((CS, CS), const),
            pl.BlockSpec((1, CS), const),
            pl.BlockSpec((k_count, CS, CS), lambda i: (0, 0, 0)),
            pl.BlockSpec((1, CS), const),
            pl.BlockSpec((CS, 4 * CS), const),
            pl.BlockSpec((1, 4 * CS), const),
            pl.BlockSpec((4 * CS, CS), const),
            pl.BlockSpec((1, CS), const),
            pl.BlockSpec((1, CS), const),
            pl.BlockSpec((1, CS), const),
        ],
        out_specs=[
            pl.BlockSpec((tile, CS), lambda i: (i, 0)),
            pl.BlockSpec((tile, VD * 3), lambda i: (i, 0)),
        ],
        out_shape=[
            jax.ShapeDtypeStruct((n, CS), jnp.float32),
            jax.ShapeDtypeStruct((n, VD * 3), jnp.float32),
        ],
    )(gp3, table_s, cproj, coords, w1n, w1d, w2m, b2, wf1, bf1, wf2, bf2,
      lng, lnb)


def kernel(scalar_feats, coords, E_idx, W_e3, b_e3, W_att1, b_att1, W_att2,
           b_att2, W_ffn1, b_ffn1, W_ffn2, b_ffn2, ln_g, ln_b):
    b, n, cs = scalar_feats.shape
    k_count = E_idx.shape[-1]
    sf = scalar_feats[0]
    co = coords[0]

    w1c = W_att1[:, 0:cs]
    w1n = W_att1[:, cs:2 * cs]
    w1d = W_att1[:, 2 * cs]
    we3s = W_e3[0:cs, :]
    wcat = jnp.concatenate([W_e3.T, (w1c @ we3s).T], axis=1)
    bcat = jnp.concatenate([b_e3, w1c @ b_e3[0:cs] + b_att1])[None, :]

    table_s, table_p, cproj = _prepass_call(sf, co, wcat, bcat, tile=1000)

    onehot = (jnp.arange(k_count)[:, None] ==
              jnp.arange(CS)[None, :]).astype(jnp.float32)
    w2m = W_att2[0][None, :, None] * onehot[:, None, :]
    b2 = jnp.broadcast_to(b_att2.reshape(1, 1), (1, CS))

    chunk = 40
    bounds = [0, 2400, 4800, 7200, n]
    ei = E_idx[0].astype(jnp.int32)
    gs = []
    for i in range(len(bounds) - 1):
        lo, hi = bounds[i], bounds[i + 1]
        ne_i = (hi - lo) * k_count
        idx_i = (jnp.transpose(ei[lo:hi]).reshape(-1)
                 .reshape(_NW, (ne_i // _NW) // chunk, chunk))
        gs.append(_make_gather(ne_i, chunk)(table_p, idx_i))

    parts = []
    for i in range(len(bounds) - 1):
        lo, hi = bounds[i], bounds[i + 1]
        sl = slice(lo, hi)
        parts.append(_main_call(
            gs[i].reshape(k_count, hi - lo, PW), table_s[sl], cproj[sl],
            co[sl], w1n.T, w1d[None, :], w2m, b2,
            W_ffn1.T, b_ffn1[None, :], W_ffn2.T, b_ffn2[None, :],
            ln_g[None, :], ln_b[None, :],
            tile=200, k_count=k_count,
        ))
    outs = jnp.concatenate([p[0] for p in parts], axis=0)
    outv = jnp.concatenate([p[1] for p in parts], axis=0)
    return outs[None], outv.reshape(1, n, VD, 3)


# speedup vs baseline: 5.7253x; 1.0872x over previous
import functools

import jax
import jax.numpy as jnp
from jax import lax
from jax.experimental import pallas as pl
from jax.experimental.pallas import tpu as pltpu
from jax.experimental.pallas import tpu_sc as plsc

CS = 128
VD = 16
PW = 128
CPW = 128


def _gelu_exact(x):
    return 0.5 * x * (1.0 + lax.erf(x * 0.7071067811865476))


def _pack16(a, b):
    ua = (lax.bitcast_convert_type(a, jnp.uint32) + jnp.uint32(0x8000)) >> jnp.uint32(16)
    ub = (lax.bitcast_convert_type(b, jnp.uint32) + jnp.uint32(0x8000)) & jnp.uint32(0xFFFF0000)
    return lax.bitcast_convert_type(ua | ub, jnp.float32)


def _unpack_lo(u):
    return lax.bitcast_convert_type(u << jnp.uint32(16), jnp.float32)


def _unpack_hi(u):
    return lax.bitcast_convert_type(u & jnp.uint32(0xFFFF0000), jnp.float32)


def _prepass_body(sf_ref, co_ref, wcat_ref, bcat_ref, ts_ref, tp_ref, cproj_ref):
    lin = jnp.dot(sf_ref[...], wcat_ref[...],
                  preferred_element_type=jnp.float32) + bcat_ref[...]
    t = sf_ref.shape[0]
    s = lin[:, 0:CS]
    v = lin[:, CS:176]
    ts_ref[...] = s
    cproj_ref[...] = lin[:, 176:304]
    tp_ref[...] = jnp.concatenate(
        [_pack16(s[:, 0:64], s[:, 64:128]),
         _pack16(v[:, 0:24], v[:, 24:48]),
         co_ref[...],
         jnp.zeros((t, PW - 91), jnp.float32)], axis=1)


def _prepass_call(sf, coords, wcat, bcat, tile):
    n = sf.shape[0]
    grid = n // tile
    return pl.pallas_call(
        _prepass_body,
        grid=(grid,),
        in_specs=[
            pl.BlockSpec((tile, CS), lambda i: (i, 0)),
            pl.BlockSpec((tile, 3), lambda i: (i, 0)),
            pl.BlockSpec((CS, 304), lambda i: (0, 0)),
            pl.BlockSpec((1, 304), lambda i: (0, 0)),
        ],
        out_specs=[
            pl.BlockSpec((tile, CS), lambda i: (i, 0)),
            pl.BlockSpec((tile, PW), lambda i: (i, 0)),
            pl.BlockSpec((tile, CPW), lambda i: (i, 0)),
        ],
        out_shape=[
            jax.ShapeDtypeStruct((n, CS), jnp.float32),
            jax.ShapeDtypeStruct((n, PW), jnp.float32),
            jax.ShapeDtypeStruct((n, CPW), jnp.float32),
        ],
    )(sf, coords, wcat, bcat)


_NC = 2
_NS = 16
_NW = _NC * _NS
_NBUF = 5


def _make_gather(ne, chunk):
    per_w = ne // _NW
    n_chunks = per_w // chunk
    mesh = plsc.VectorSubcoreMesh(core_axis_name="c", subcore_axis_name="s")

    @functools.partial(
        pl.kernel, mesh=mesh,
        out_type=jax.ShapeDtypeStruct((ne, PW), jnp.float32),
        scratch_types=[
            pltpu.VMEM((n_chunks, chunk), jnp.int32),
            pltpu.VMEM((_NBUF, chunk, PW), jnp.float32),
        ] + [pltpu.SemaphoreType.DMA] * (2 * _NBUF),
    )
    def gather(tp_hbm, idx_hbm, outp_hbm, idx_v, bufs, *sems):
        gsem = sems[:_NBUF]
        ssem = sems[_NBUF:]
        wid = lax.axis_index("s") * _NC + lax.axis_index("c")
        base = wid * per_w

        pltpu.sync_copy(idx_hbm.at[wid], idx_v)

        pend_g = [None] * _NBUF
        pend_s = [None] * _NBUF
        for c in range(n_chunks + _NBUF - 1):
            if c < n_chunks:
                b = c % _NBUF
                if pend_s[b] is not None:
                    pend_s[b].wait()
                pend_g[b] = pltpu.async_copy(
                    tp_hbm.at[idx_v.at[c]], bufs.at[b], gsem[b])
            d = c - (_NBUF - 1)
            if d >= 0:
                bd = d % _NBUF
                pend_g[bd].wait()
                off = pl.multiple_of(base + d * chunk, 8)
                pend_s[bd] = pltpu.async_copy(
                    bufs.at[bd], outp_hbm.at[pl.ds(off, chunk)], ssem[bd])
        for b in range(_NBUF):
            if pend_s[b] is not None:
                pend_s[b].wait()

    return gather


def _main_body(gp_ref, tab_ref, cp_ref, co_ref, w1n_ref, w1d_ref,
               w2m_ref, b2_ref, wf1_ref, bf1_ref, wf2_ref, bf2_ref, lng_ref,
               lnb_ref, outs_ref, outv_ref, k_count):
    t = tab_ref.shape[0]
    cc = co_ref[...]
    cp = cp_ref[...]
    w1n_lo = w1n_ref[0:64, :]
    w1n_hi = w1n_ref[64:CS, :]
    w1d = w1d_ref[...]

    c4 = jnp.concatenate([cc, jnp.zeros((t, 1), jnp.float32)], axis=1)

    lacc = jnp.zeros((t, CS), jnp.float32)
    for k in range(k_count):
        row = gp_ref[k]
        u = lax.bitcast_convert_type(row[:, 0:64], jnp.uint32)
        s_lo = _unpack_lo(u)
        s_hi = _unpack_hi(u)
        diff = c4 - row[:, 88:92]
        dist = jnp.sqrt(jnp.sum(diff * diff, axis=1, keepdims=True) + 1e-6)
        h = (cp + jnp.dot(s_lo, w1n_lo, preferred_element_type=jnp.float32)
             + jnp.dot(s_hi, w1n_hi, preferred_element_type=jnp.float32)
             + dist * w1d)
        h = _gelu_exact(h)
        lacc = lacc + jnp.dot(h, w2m_ref[k], preferred_element_type=jnp.float32)

    logits = jnp.clip(lacc[:, 0:k_count] + b2_ref[:, 0:k_count], -10000.0, 10.0)
    m = jnp.max(logits, axis=1, keepdims=True)
    e = jnp.exp(logits - m)
    att = e / jnp.sum(e, axis=1, keepdims=True)

    smsg_lo = jnp.zeros((t, 64), jnp.float32)
    smsg_hi = jnp.zeros((t, 64), jnp.float32)
    vmsg_lo = jnp.zeros((t, 24), jnp.float32)
    vmsg_hi = jnp.zeros((t, 24), jnp.float32)
    for k in range(k_count):
        row = gp_ref[k]
        a = att[:, k:k + 1]
        us = lax.bitcast_convert_type(row[:, 0:64], jnp.uint32)
        uv = lax.bitcast_convert_type(row[:, 64:88], jnp.uint32)
        smsg_lo = smsg_lo + a * _unpack_lo(us)
        smsg_hi = smsg_hi + a * _unpack_hi(us)
        vmsg_lo = vmsg_lo + a * _unpack_lo(uv)
        vmsg_hi = vmsg_hi + a * _unpack_hi(uv)

    smsg = jnp.concatenate([smsg_lo, smsg_hi], axis=1)
    x = tab_ref[...] + smsg
    mu = jnp.mean(x, axis=1, keepdims=True)
    xc = x - mu
    var = jnp.mean(xc * xc, axis=1, keepdims=True)
    x = xc * jax.lax.rsqrt(var + 1e-5) * lng_ref[...] + lnb_ref[...]

    f = jnp.dot(x, wf1_ref[...], preferred_element_type=jnp.float32) + bf1_ref[...]
    f = _gelu_exact(f)
    f = jnp.dot(f, wf2_ref[...], preferred_element_type=jnp.float32) + bf2_ref[...]

    outs_ref[...] = x + f
    outv_ref[...] = jnp.concatenate([vmsg_lo, vmsg_hi], axis=1)


def _main_call(gp3, table_s, cproj, coords, w1n, w1d, w2m, b2, wf1, bf1,
               wf2, bf2, lng, lnb, tile, k_count):
    n = table_s.shape[0]
    grid = n // tile
    const = lambda i: (0, 0)
    return pl.pallas_call(
        functools.partial(_main_body, k_count=k_count),
        grid=(grid,),
        in_specs=[
            pl.BlockSpec((k_count, tile, PW), lambda i: (0, i, 0)),
            pl.BlockSpec((tile, CS), lambda i: (i, 0)),
            pl.BlockSpec((tile, CPW), lambda i: (i, 0)),
            pl.BlockSpec((tile, 3), lambda i: (i, 0)),
            pl.BlockSpec((CS, CS), const),
            pl.BlockSpec((1, CS), const),
            pl.BlockSpec((k_count, CS, CS), lambda i: (0, 0, 0)),
            pl.BlockSpec((1, CS), const),
            pl.BlockSpec((CS, 4 * CS), const),
            pl.BlockSpec((1, 4 * CS), const),
            pl.BlockSpec((4 * CS, CS), const),
            pl.BlockSpec((1, CS), const),
            pl.BlockSpec((1, CS), const),
            pl.BlockSpec((1, CS), const),
        ],
        out_specs=[
            pl.BlockSpec((tile, CS), lambda i: (i, 0)),
            pl.BlockSpec((tile, VD * 3), lambda i: (i, 0)),
        ],
        out_shape=[
            jax.ShapeDtypeStruct((n, CS), jnp.float32),
            jax.ShapeDtypeStruct((n, VD * 3), jnp.float32),
        ],
    )(gp3, table_s, cproj, coords, w1n, w1d, w2m, b2, wf1, bf1, wf2, bf2,
      lng, lnb)


def kernel(scalar_feats, coords, E_idx, W_e3, b_e3, W_att1, b_att1, W_att2,
           b_att2, W_ffn1, b_ffn1, W_ffn2, b_ffn2, ln_g, ln_b):
    b, n, cs = scalar_feats.shape
    k_count = E_idx.shape[-1]
    sf = scalar_feats[0]
    co = coords[0]

    w1c = W_att1[:, 0:cs]
    w1n = W_att1[:, cs:2 * cs]
    w1d = W_att1[:, 2 * cs]
    we3s = W_e3[0:cs, :]
    wcat = jnp.concatenate([W_e3.T, (w1c @ we3s).T], axis=1)
    bcat = jnp.concatenate([b_e3, w1c @ b_e3[0:cs] + b_att1])[None, :]

    table_s, table_p, cproj = _prepass_call(sf, co, wcat, bcat, tile=1000)

    onehot = (jnp.arange(k_count)[:, None] ==
              jnp.arange(CS)[None, :]).astype(jnp.float32)
    w2m = W_att2[0][None, :, None] * onehot[:, None, :]
    b2 = jnp.broadcast_to(b_att2.reshape(1, 1), (1, CS))

    chunk = 40
    bounds = [0, 2400, 4800, 7200, n]
    ei = E_idx[0].astype(jnp.int32)
    gs = []
    for i in range(len(bounds) - 1):
        lo, hi = bounds[i], bounds[i + 1]
        ne_i = (hi - lo) * k_count
        idx_i = (jnp.transpose(ei[lo:hi]).reshape(-1)
                 .reshape(_NW, (ne_i // _NW) // chunk, chunk))
        gs.append(_make_gather(ne_i, chunk)(table_p, idx_i))

    parts = []
    for i in range(len(bounds) - 1):
        lo, hi = bounds[i], bounds[i + 1]
        sl = slice(lo, hi)
        parts.append(_main_call(
            gs[i].reshape(k_count, hi - lo, PW), table_s[sl], cproj[sl],
            co[sl], w1n.T, w1d[None, :], w2m, b2,
            W_ffn1.T, b_ffn1[None, :], W_ffn2.T, b_ffn2[None, :],
            ln_g[None, :], ln_b[None, :],
            tile=200, k_count=k_count,
        ))
    outs = jnp.concatenate([p[0] for p in parts], axis=0)
    outv = jnp.concatenate([p[1] for p in parts], axis=0)
    return outs[None], outv.reshape(1, n, VD, 3)


# speedup vs baseline: 8.4883x; 1.4826x over previous
import functools

import jax
import jax.numpy as jnp
from jax import lax
from jax.experimental import pallas as pl
from jax.experimental.pallas import tpu as pltpu
from jax.experimental.pallas import tpu_sc as plsc

CS = 128
VD = 16
PW = 128
CPW = 128


def _gelu_exact(x):
    return 0.5 * x * (1.0 + lax.erf(x * 0.7071067811865476))


def _pack16(a, b):
    ua = (lax.bitcast_convert_type(a, jnp.uint32) + jnp.uint32(0x8000)) >> jnp.uint32(16)
    ub = (lax.bitcast_convert_type(b, jnp.uint32) + jnp.uint32(0x8000)) & jnp.uint32(0xFFFF0000)
    return lax.bitcast_convert_type(ua | ub, jnp.float32)


def _unpack_lo(u):
    return lax.bitcast_convert_type(u << jnp.uint32(16), jnp.float32)


def _unpack_hi(u):
    return lax.bitcast_convert_type(u & jnp.uint32(0xFFFF0000), jnp.float32)


def _prepass_body(sf_ref, co_ref, wcat_ref, bcat_ref, ts_ref, tp_ref, cproj_ref):
    lin = jnp.dot(sf_ref[...], wcat_ref[...],
                  preferred_element_type=jnp.float32) + bcat_ref[...]
    t = sf_ref.shape[0]
    s = lin[:, 0:CS]
    v = lin[:, CS:176]
    ts_ref[...] = s
    cproj_ref[...] = lin[:, 176:304]
    tp_ref[...] = jnp.concatenate(
        [_pack16(s[:, 0:64], s[:, 64:128]),
         _pack16(v[:, 0:24], v[:, 24:48]),
         co_ref[...],
         jnp.zeros((t, PW - 91), jnp.float32)], axis=1)


def _prepass_call(sf, coords, wcat, bcat, tile):
    n = sf.shape[0]
    grid = n // tile
    return pl.pallas_call(
        _prepass_body,
        grid=(grid,),
        in_specs=[
            pl.BlockSpec((tile, CS), lambda i: (i, 0)),
            pl.BlockSpec((tile, 3), lambda i: (i, 0)),
            pl.BlockSpec((CS, 304), lambda i: (0, 0)),
            pl.BlockSpec((1, 304), lambda i: (0, 0)),
        ],
        out_specs=[
            pl.BlockSpec((tile, CS), lambda i: (i, 0)),
            pl.BlockSpec((tile, PW), lambda i: (i, 0)),
            pl.BlockSpec((tile, CPW), lambda i: (i, 0)),
        ],
        out_shape=[
            jax.ShapeDtypeStruct((n, CS), jnp.float32),
            jax.ShapeDtypeStruct((n, PW), jnp.float32),
            jax.ShapeDtypeStruct((n, CPW), jnp.float32),
        ],
    )(sf, coords, wcat, bcat)


_NC = 2
_NS = 16
_NW = _NC * _NS
_NBUF = 5


def _make_gather(ne, chunk):
    per_w = ne // _NW
    n_chunks = per_w // chunk
    mesh = plsc.VectorSubcoreMesh(core_axis_name="c", subcore_axis_name="s")

    @functools.partial(
        pl.kernel, mesh=mesh,
        out_type=jax.ShapeDtypeStruct((ne, PW), jnp.float32),
        scratch_types=[
            pltpu.VMEM((n_chunks, chunk), jnp.int32),
            pltpu.VMEM((_NBUF, chunk, PW), jnp.float32),
        ] + [pltpu.SemaphoreType.DMA] * (2 * _NBUF),
    )
    def gather(tp_hbm, idx_hbm, outp_hbm, idx_v, bufs, *sems):
        gsem = sems[:_NBUF]
        ssem = sems[_NBUF:]
        wid = lax.axis_index("s") * _NC + lax.axis_index("c")
        base = wid * per_w

        pltpu.sync_copy(idx_hbm.at[wid], idx_v)

        pend_g = [None] * _NBUF
        pend_s = [None] * _NBUF
        for c in range(n_chunks + _NBUF - 1):
            if c < n_chunks:
                b = c % _NBUF
                if pend_s[b] is not None:
                    pend_s[b].wait()
                pend_g[b] = pltpu.async_copy(
                    tp_hbm.at[idx_v.at[c]], bufs.at[b], gsem[b])
            d = c - (_NBUF - 1)
            if d >= 0:
                bd = d % _NBUF
                pend_g[bd].wait()
                off = pl.multiple_of(base + d * chunk, 8)
                pend_s[bd] = pltpu.async_copy(
                    bufs.at[bd], outp_hbm.at[pl.ds(off, chunk)], ssem[bd])
        for b in range(_NBUF):
            if pend_s[b] is not None:
                pend_s[b].wait()

    return gather


def _main_body(gp_ref, tab_ref, cp_ref, co_ref, w1n_ref, w1d_ref,
               w2m_ref, b2_ref, wf1_ref, bf1_ref, wf2_ref, bf2_ref, lng_ref,
               lnb_ref, outs_ref, outv_ref, k_count):
    t = tab_ref.shape[0]
    cc = co_ref[...]
    cp = cp_ref[...]
    w1n_lo = w1n_ref[0:64, :]
    w1n_hi = w1n_ref[64:CS, :]
    w1d = w1d_ref[...]

    c4 = jnp.concatenate([cc, jnp.zeros((t, 1), jnp.float32)], axis=1)

    lacc = jnp.zeros((t, CS), jnp.float32)
    for k in range(k_count):
        row = gp_ref[k]
        u = lax.bitcast_convert_type(row[:, 0:64], jnp.uint32)
        s_lo = _unpack_lo(u)
        s_hi = _unpack_hi(u)
        diff = c4 - row[:, 88:92]
        dist = jnp.sqrt(jnp.sum(diff * diff, axis=1, keepdims=True) + 1e-6)
        h = (cp + jnp.dot(s_lo, w1n_lo, preferred_element_type=jnp.float32)
             + jnp.dot(s_hi, w1n_hi, preferred_element_type=jnp.float32)
             + dist * w1d)
        h = _gelu_exact(h)
        lacc = lacc + jnp.dot(h, w2m_ref[k], preferred_element_type=jnp.float32)

    logits = jnp.clip(lacc[:, 0:k_count] + b2_ref[:, 0:k_count], -10000.0, 10.0)
    m = jnp.max(logits, axis=1, keepdims=True)
    e = jnp.exp(logits - m)
    att = e / jnp.sum(e, axis=1, keepdims=True)

    smsg_lo = jnp.zeros((t, 64), jnp.float32)
    smsg_hi = jnp.zeros((t, 64), jnp.float32)
    vmsg_lo = jnp.zeros((t, 24), jnp.float32)
    vmsg_hi = jnp.zeros((t, 24), jnp.float32)
    for k in range(k_count):
        row = gp_ref[k]
        a = att[:, k:k + 1]
        us = lax.bitcast_convert_type(row[:, 0:64], jnp.uint32)
        uv = lax.bitcast_convert_type(row[:, 64:88], jnp.uint32)
        smsg_lo = smsg_lo + a * _unpack_lo(us)
        smsg_hi = smsg_hi + a * _unpack_hi(us)
        vmsg_lo = vmsg_lo + a * _unpack_lo(uv)
        vmsg_hi = vmsg_hi + a * _unpack_hi(uv)

    smsg = jnp.concatenate([smsg_lo, smsg_hi], axis=1)
    x = tab_ref[...] + smsg
    mu = jnp.mean(x, axis=1, keepdims=True)
    xc = x - mu
    var = jnp.mean(xc * xc, axis=1, keepdims=True)
    x = xc * jax.lax.rsqrt(var + 1e-5) * lng_ref[...] + lnb_ref[...]

    f = jnp.dot(x, wf1_ref[...], preferred_element_type=jnp.float32) + bf1_ref[...]
    f = _gelu_exact(f)
    f = jnp.dot(f, wf2_ref[...], preferred_element_type=jnp.float32) + bf2_ref[...]

    outs_ref[...] = x + f
    outv_ref[...] = jnp.concatenate([vmsg_lo, vmsg_hi], axis=1)


def _main_call(gp3, table_s, cproj, coords, w1n, w1d, w2m, b2, wf1, bf1,
               wf2, bf2, lng, lnb, tile, k_count):
    n = table_s.shape[0]
    grid = n // tile
    const = lambda i: (0, 0)
    return pl.pallas_call(
        functools.partial(_main_body, k_count=k_count),
        grid=(grid,),
        in_specs=[
            pl.BlockSpec((k_count, tile, PW), lambda i: (0, i, 0)),
            pl.BlockSpec((tile, CS), lambda i: (i, 0)),
            pl.BlockSpec((tile, CPW), lambda i: (i, 0)),
            pl.BlockSpec((tile, 3), lambda i: (i, 0)),
            pl.BlockSpec((CS, CS), const),
            pl.BlockSpec((1, CS), const),
            pl.BlockSpec((k_count, CS, CS), lambda i: (0, 0, 0)),
            pl.BlockSpec((1, CS), const),
            pl.BlockSpec((CS, 4 * CS), const),
            pl.BlockSpec((1, 4 * CS), const),
            pl.BlockSpec((4 * CS, CS), const),
            pl.BlockSpec((1, CS), const),
            pl.BlockSpec((1, CS), const),
            pl.BlockSpec((1, CS), const),
        ],
        out_specs=[
            pl.BlockSpec((tile, CS), lambda i: (i, 0)),
            pl.BlockSpec((tile, VD * 3), lambda i: (i, 0)),
        ],
        out_shape=[
            jax.ShapeDtypeStruct((n, CS), jnp.float32),
            jax.ShapeDtypeStruct((n, VD * 3), jnp.float32),
        ],
    )(gp3, table_s, cproj, coords, w1n, w1d, w2m, b2, wf1, bf1, wf2, bf2,
      lng, lnb)


def kernel(scalar_feats, coords, E_idx, W_e3, b_e3, W_att1, b_att1, W_att2,
           b_att2, W_ffn1, b_ffn1, W_ffn2, b_ffn2, ln_g, ln_b):
    b, n, cs = scalar_feats.shape
    k_count = E_idx.shape[-1]
    sf = scalar_feats[0]
    co = coords[0]

    w1c = W_att1[:, 0:cs]
    w1n = W_att1[:, cs:2 * cs]
    w1d = W_att1[:, 2 * cs]
    we3s = W_e3[0:cs, :]
    wcat = jnp.concatenate([W_e3.T, (w1c @ we3s).T], axis=1)
    bcat = jnp.concatenate([b_e3, w1c @ b_e3[0:cs] + b_att1])[None, :]

    table_s, table_p, cproj = _prepass_call(sf, co, wcat, bcat, tile=1000)

    onehot = (jnp.arange(k_count)[:, None] ==
              jnp.arange(CS)[None, :]).astype(jnp.float32)
    w2m = W_att2[0][None, :, None] * onehot[:, None, :]
    b2 = jnp.broadcast_to(b_att2.reshape(1, 1), (1, CS))

    chunk = 40
    bounds = [0, 2400, 4800, 7200, n]
    ei = E_idx[0].astype(jnp.int32)
    gs = []
    for i in range(len(bounds) - 1):
        lo, hi = bounds[i], bounds[i + 1]
        ne_i = (hi - lo) * k_count
        idx_i = (jnp.transpose(ei[lo:hi]).reshape(-1)
                 .reshape(_NW, (ne_i // _NW) // chunk, chunk))
        gs.append(_make_gather(ne_i, chunk)(table_p, idx_i))

    parts = []
    for i in range(len(bounds) - 1):
        lo, hi = bounds[i], bounds[i + 1]
        sl = slice(lo, hi)
        parts.append(_main_call(
            gs[i].reshape(k_count, hi - lo, PW), table_s[sl], cproj[sl],
            co[sl], w1n.T, w1d[None, :], w2m, b2,
            W_ffn1.T, b_ffn1[None, :], W_ffn2.T, b_ffn2[None, :],
            ln_g[None, :], ln_b[None, :],
            tile=400, k_count=k_count,
        ))
    outs = jnp.concatenate([p[0] for p in parts], axis=0)
    outv = jnp.concatenate([p[1] for p in parts], axis=0)
    return outs[None], outv.reshape(1, n, VD, 3)
